# baseline (speedup 1.0000x reference)
import functools
import math

import jax
import jax.numpy as jnp
from jax import lax
from jax.experimental import pallas as pl
from jax.experimental.pallas import tpu as pltpu
from jax.experimental.pallas import tpu_sc as plsc

VOCAB = 100000
D = 128
SCALE = math.sqrt(float(D))

_INFO = plsc.get_sparse_core_info()
_NC = _INFO.num_cores
_NS = _INFO.num_subcores
_NW = _NC * _NS

CHUNK = 128


NBUF = 5
LOOK = 3


@functools.cache
def _gather_kernel(B):
    n_chunks = B // _NW // CHUNK
    per_w = n_chunks * CHUNK
    assert n_chunks % NBUF == 0 and n_chunks > 2 * NBUF
    mesh = plsc.VectorSubcoreMesh(core_axis_name="c", subcore_axis_name="s")

    @functools.partial(
        pl.kernel,
        mesh=mesh,
        out_type=jax.ShapeDtypeStruct((B, D), jnp.float32),
        scratch_types=(
            [pltpu.VMEM((n_chunks, CHUNK), jnp.int32)]
            + [pltpu.VMEM((CHUNK, D), jnp.float32) for _ in range(NBUF)]
            + [pltpu.SemaphoreType.DMA for _ in range(2 * NBUF)]
        ),
    )
    def k(idx_hbm, table_hbm, out_hbm, idx_v, *rest):
        rows = rest[:NBUF]
        gsem = rest[NBUF:2 * NBUF]
        wsem = rest[2 * NBUF:3 * NBUF]
        wid = lax.axis_index("s") * _NC + lax.axis_index("c")
        pltpu.sync_copy(idx_hbm.at[wid], idx_v)
        base = wid * per_w

        def start_gather(c, b):
            pltpu.async_copy(table_hbm.at[idx_v.at[c]], rows[b], gsem[b])

        def wait_gather(c, b):
            pltpu.make_async_copy(table_hbm.at[idx_v.at[c]], rows[b],
                                  gsem[b]).wait()

        def scale_rows(b):
            r = rows[b]
            unroll = 2

            def srow(i, carry):
                for u in range(unroll):
                    for j in range(D // 16):
                        sl = (i * unroll + u, pl.ds(j * 16, 16))
                        r[sl] = r[sl] * SCALE
                return carry

            lax.fori_loop(0, CHUNK // unroll, srow, 0)

        def start_write(c, b):
            pltpu.async_copy(rows[b], out_hbm.at[pl.ds(base + c * CHUNK, CHUNK)],
                             wsem[b])

        def wait_write(c, b):
            pltpu.make_async_copy(rows[b],
                                  out_hbm.at[pl.ds(base + c * CHUNK, CHUNK)],
                                  wsem[b]).wait()

        for c in range(LOOK):
            start_gather(c, c)
        for g in range(NBUF):
            wait_gather(g, g)
            scale_rows(g)
            start_write(g, g)
            if g + LOOK >= NBUF:
                wait_write(g + LOOK - NBUF, (g + LOOK) % NBUF)
            start_gather(g + LOOK, (g + LOOK) % NBUF)

        def body(outer, carry):
            for b in range(NBUF):
                g = outer * NBUF + b
                wait_gather(g, b)
                scale_rows(b)
                start_write(g, b)
                b2 = (b + LOOK) % NBUF
                wait_write(g + LOOK - NBUF, b2)
                start_gather(g + LOOK, b2)
            return carry

        lax.fori_loop(1, n_chunks // NBUF - 1, body, 0)

        for g in range(n_chunks - NBUF, n_chunks):
            b = g % NBUF
            wait_gather(g, b)
            scale_rows(b)
            start_write(g, b)
            if g + LOOK < n_chunks:
                b2 = (g + LOOK) % NBUF
                wait_write(g + LOOK - NBUF, b2)
                start_gather(g + LOOK, b2)
        for g in range(n_chunks - NBUF, n_chunks):
            wait_write(g, g % NBUF)

    return k


def kernel(tokens, table):
    b, h = tokens.shape
    B = b * h
    idx = tokens.reshape(_NW, B // _NW // CHUNK, CHUNK)
    out = _gather_kernel(B)(idx, table)
    return out.reshape(b, h, D)


# speedup vs baseline: 1.0006x; 1.0006x over previous
import functools
import math

import jax
import jax.numpy as jnp
from jax import lax
from jax.experimental import pallas as pl
from jax.experimental.pallas import tpu as pltpu
from jax.experimental.pallas import tpu_sc as plsc

VOCAB = 100000
D = 128
SCALE = math.sqrt(float(D))

_INFO = plsc.get_sparse_core_info()
_NC = _INFO.num_cores
_NS = _INFO.num_subcores
_NW = _NC * _NS

CHUNK = 128


NSLOT = 6
NPAR = NSLOT // 2


@functools.cache
def _gather_kernel(B):
    n_chunks = B // _NW // CHUNK
    n_pairs = n_chunks // 2
    per_w = n_chunks * CHUNK
    assert n_chunks % 2 == 0 and (n_pairs - 4 - 3) % NPAR == 0 and n_pairs > 8
    mesh = plsc.VectorSubcoreMesh(core_axis_name="c", subcore_axis_name="s")

    @functools.partial(
        pl.kernel,
        mesh=mesh,
        out_type=jax.ShapeDtypeStruct((B, D), jnp.float32),
        scratch_types=(
            [pltpu.VMEM((n_chunks, CHUNK), jnp.int32),
             pltpu.VMEM((NSLOT * CHUNK, D), jnp.float32)]
            + [pltpu.SemaphoreType.DMA for _ in range(NSLOT + NPAR)]
        ),
    )
    def k(idx_hbm, table_hbm, out_hbm, idx_v, rows_v, *sems):
        gsem = sems[:NSLOT]
        wsem = sems[NSLOT:NSLOT + NPAR]
        wid = lax.axis_index("s") * _NC + lax.axis_index("c")
        pltpu.sync_copy(idx_hbm.at[wid], idx_v)
        base = wid * per_w

        def slot(s, n=1):
            return rows_v.at[pl.ds(s * CHUNK, n * CHUNK)]

        def start_gather(c, s):
            pltpu.async_copy(table_hbm.at[idx_v.at[c]], slot(s), gsem[s])

        def wait_gather(c, s):
            pltpu.make_async_copy(table_hbm.at[idx_v.at[c]], slot(s),
                                  gsem[s]).wait()

        def scale_slot(s):
            unroll = 2

            def srow(i, carry):
                for u in range(unroll):
                    for j in range(D // 16):
                        sl = (s * CHUNK + i * unroll + u, pl.ds(j * 16, 16))
                        rows_v[sl] = rows_v[sl] * SCALE
                return carry

            lax.fori_loop(0, CHUNK // unroll, srow, 0)

        def start_write(p, q):
            pltpu.async_copy(slot(2 * q, 2),
                             out_hbm.at[pl.ds(base + p * 2 * CHUNK, 2 * CHUNK)],
                             wsem[q])

        def wait_write(p, q):
            pltpu.make_async_copy(slot(2 * q, 2),
                                  out_hbm.at[pl.ds(base + p * 2 * CHUNK,
                                                   2 * CHUNK)],
                                  wsem[q]).wait()

        def do_pair(p, q, refill, guard_wait):
            wait_gather(2 * p, 2 * q)
            scale_slot(2 * q)
            wait_gather(2 * p + 1, 2 * q + 1)
            scale_slot(2 * q + 1)
            start_write(p, q)
            if refill:
                q2 = (q + 2) % NPAR
                if guard_wait:
                    wait_write(p - 1, q2)
                start_gather(2 * (p + 2), 2 * q2)
                start_gather(2 * (p + 2) + 1, 2 * q2 + 1)

        for c in range(4):
            start_gather(c, c)
        do_pair(0, 0, refill=True, guard_wait=False)
        do_pair(1, 1, refill=True, guard_wait=True)
        do_pair(2, 2, refill=True, guard_wait=True)

        def body(outer, carry):
            for q in range(NPAR):
                p = outer * NPAR + q
                do_pair(p, q, refill=True, guard_wait=True)
            return carry

        lax.fori_loop(1, (n_pairs - 4 - 3) // NPAR + 1, body, 0)

        for p in range(n_pairs - 4, n_pairs):
            do_pair(p, p % NPAR, refill=(p + 2 < n_pairs), guard_wait=True)
        for p in range(n_pairs - NPAR, n_pairs):
            wait_write(p, p % NPAR)

    return k


def kernel(tokens, table):
    b, h = tokens.shape
    B = b * h
    idx = tokens.reshape(_NW, B // _NW // CHUNK, CHUNK)
    out = _gather_kernel(B)(idx, table)
    return out.reshape(b, h, D)


# speedup vs baseline: 1.0023x; 1.0017x over previous
import functools
import math

import jax
import jax.numpy as jnp
from jax import lax
from jax.experimental import pallas as pl
from jax.experimental.pallas import tpu as pltpu
from jax.experimental.pallas import tpu_sc as plsc

VOCAB = 100000
D = 128
SCALE = math.sqrt(float(D))

_INFO = plsc.get_sparse_core_info()
_NC = _INFO.num_cores
_NS = _INFO.num_subcores
_NW = _NC * _NS

CHUNK = 128


NBUF = 5
LOOK = 3


@functools.cache
def _gather_kernel(B):
    n_chunks = B // _NW // CHUNK
    per_w = n_chunks * CHUNK
    assert n_chunks % NBUF == 0 and n_chunks > 2 * NBUF
    mesh = plsc.VectorSubcoreMesh(core_axis_name="c", subcore_axis_name="s")

    @functools.partial(
        pl.kernel,
        mesh=mesh,
        out_type=jax.ShapeDtypeStruct((B, D), jnp.float32),
        scratch_types=(
            [pltpu.VMEM((n_chunks, CHUNK), jnp.int32)]
            + [pltpu.VMEM((CHUNK, D), jnp.float32) for _ in range(NBUF)]
            + [pltpu.SemaphoreType.DMA for _ in range(2 * NBUF)]
        ),
    )
    def k(idx_hbm, table_hbm, out_hbm, idx_v, *rest):
        rows = rest[:NBUF]
        gsem = rest[NBUF:2 * NBUF]
        wsem = rest[2 * NBUF:3 * NBUF]
        wid = lax.axis_index("s") * _NC + lax.axis_index("c")
        pltpu.sync_copy(idx_hbm.at[wid], idx_v)
        base = wid * per_w

        def start_gather(c, b):
            pltpu.async_copy(table_hbm.at[idx_v.at[c]], rows[b], gsem[b])

        def wait_gather(c, b):
            pltpu.make_async_copy(table_hbm.at[idx_v.at[c]], rows[b],
                                  gsem[b]).wait()

        def scale_rows(b):
            r = rows[b]
            unroll = 2

            def srow(i, carry):
                for u in range(unroll):
                    for j in range(D // 16):
                        sl = (i * unroll + u, pl.ds(j * 16, 16))
                        r[sl] = r[sl] * SCALE
                return carry

            lax.fori_loop(0, CHUNK // unroll, srow, 0)

        def start_write(c, b):
            pltpu.async_copy(rows[b], out_hbm.at[pl.ds(base + c * CHUNK, CHUNK)],
                             wsem[b])

        def wait_write(c, b):
            pltpu.make_async_copy(rows[b],
                                  out_hbm.at[pl.ds(base + c * CHUNK, CHUNK)],
                                  wsem[b]).wait()

        for c in range(LOOK):
            start_gather(c, c)
        for g in range(NBUF):
            wait_gather(g, g)
            scale_rows(g)
            start_write(g, g)
            if g + LOOK >= NBUF:
                wait_write(g + LOOK - NBUF, (g + LOOK) % NBUF)
            start_gather(g + LOOK, (g + LOOK) % NBUF)

        def body(outer, carry):
            for b in range(NBUF):
                g = outer * NBUF + b
                wait_gather(g, b)
                scale_rows(b)
                start_write(g, b)
                b2 = (b + LOOK) % NBUF
                wait_write(g + LOOK - NBUF, b2)
                start_gather(g + LOOK, b2)
            return carry

        lax.fori_loop(1, n_chunks // NBUF - 1, body, 0)

        for g in range(n_chunks - NBUF, n_chunks):
            b = g % NBUF
            wait_gather(g, b)
            scale_rows(b)
            start_write(g, b)
            if g + LOOK < n_chunks:
                b2 = (g + LOOK) % NBUF
                wait_write(g + LOOK - NBUF, b2)
                start_gather(g + LOOK, b2)
        for g in range(n_chunks - NBUF, n_chunks):
            wait_write(g, g % NBUF)

    return k


def kernel(tokens, table):
    b, h = tokens.shape
    B = b * h
    idx = tokens.reshape(_NW, B // _NW // CHUNK, CHUNK)
    out = _gather_kernel(B)(idx, table)
    return out.reshape(b, h, D)
